# Initial kernel scaffold; baseline (speedup 1.0000x reference)
import jax
import jax.numpy as jnp
from jax.experimental import pallas as pl


def kernel(inputs, word_table, pos_table):
    raise NotImplementedError("write your pallas kernel here")


# speedup vs baseline: 3.6958x; 3.6958x over previous
import functools

import jax
import jax.numpy as jnp
from jax import lax
from jax.experimental import pallas as pl
from jax.experimental.pallas import tpu as pltpu
from jax.experimental.pallas import tpu_sc as plsc

SEQ_LEN = 200
VOCAB = 100000
D = 64
BATCH = 4096

NUM_WORKERS = 32
SEQ_PER_WORKER = BATCH // NUM_WORKERS
CHUNK_SEQ = 4
CHUNK_ROWS = CHUNK_SEQ * SEQ_LEN
NUM_CHUNKS = SEQ_PER_WORKER // CHUNK_SEQ
GATHER_SPLIT = 10
GATHER_ROWS = CHUNK_ROWS // GATHER_SPLIT
ROWS_PER_WORKER = SEQ_PER_WORKER * SEQ_LEN


def _body(idx_hbm, table_hbm, pos_hbm, out_hbm, idx_v, g_v, pos_v, sem):
    wid = lax.axis_index("s") * 2 + lax.axis_index("c")
    worker_base = wid * ROWS_PER_WORKER

    pltpu.sync_copy(pos_hbm, pos_v)

    def chunk_body(c, carry):
        row_base = worker_base + c * CHUNK_ROWS

        pltpu.sync_copy(idx_hbm.at[pl.ds(row_base, CHUNK_ROWS)], idx_v)

        copies = []
        for j in range(GATHER_SPLIT):
            copies.append(pltpu.async_copy(
                table_hbm.at[idx_v.at[pl.ds(j * GATHER_ROWS, GATHER_ROWS)]],
                g_v.at[pl.ds(j * GATHER_ROWS, GATHER_ROWS)],
                sem,
            ))
        for cp in copies:
            cp.wait()

        def add_body(l, carry2):
            for r in range(D // 16):
                pv = pos_v[pl.ds(l * D + r * 16, 16)]
                for s in range(CHUNK_SEQ):
                    row = s * SEQ_LEN + l
                    g_v[row, pl.ds(r * 16, 16)] = g_v[row, pl.ds(r * 16, 16)] + pv
            return carry2

        lax.fori_loop(0, SEQ_LEN, add_body, 0)

        pltpu.sync_copy(g_v, out_hbm.at[pl.ds(row_base, CHUNK_ROWS)])
        return carry

    lax.fori_loop(0, NUM_CHUNKS, chunk_body, 0)


@jax.jit
def _pos_embed(flat_idx, word_table, pos_flat):
    mesh = plsc.VectorSubcoreMesh(core_axis_name="c", subcore_axis_name="s")
    return pl.kernel(
        _body,
        mesh=mesh,
        compiler_params=pltpu.CompilerParams(use_tc_tiling_on_sc=False),
        out_type=jax.ShapeDtypeStruct((BATCH * SEQ_LEN, D), jnp.float32),
        scratch_types=[
            pltpu.VMEM((CHUNK_ROWS,), jnp.int32),
            pltpu.VMEM((CHUNK_ROWS, D), jnp.float32),
            pltpu.VMEM((SEQ_LEN * D,), jnp.float32),
            pltpu.SemaphoreType.DMA,
        ],
    )(flat_idx, word_table, pos_flat)


def kernel(inputs, word_table, pos_table):
    flat_idx = inputs.reshape(-1)
    pos_flat = pos_table.reshape(-1)
    out = _pos_embed(flat_idx, word_table, pos_flat)
    return out.reshape(BATCH, SEQ_LEN, D)


# speedup vs baseline: 3.8247x; 1.0349x over previous
import functools

import jax
import jax.numpy as jnp
from jax import lax
from jax.experimental import pallas as pl
from jax.experimental.pallas import tpu as pltpu
from jax.experimental.pallas import tpu_sc as plsc

SEQ_LEN = 200
VOCAB = 100000
D = 64
BATCH = 4096

NUM_WORKERS = 32
SEQ_PER_WORKER = BATCH // NUM_WORKERS
CHUNK_SEQ = 4
CHUNK_ROWS = CHUNK_SEQ * SEQ_LEN
NUM_CHUNKS = SEQ_PER_WORKER // CHUNK_SEQ
GATHER_SPLIT = 10
GATHER_ROWS = CHUNK_ROWS // GATHER_SPLIT
ROWS_PER_WORKER = SEQ_PER_WORKER * SEQ_LEN


def _body(idx_hbm, table_hbm, pos_hbm, out_hbm, idx_v, g_v, pos_v, gsem, osem):
    wid = lax.axis_index("s") * 2 + lax.axis_index("c")
    worker_base = wid * ROWS_PER_WORKER

    pltpu.sync_copy(pos_hbm, pos_v)

    def fire_chunk(c, p):
        row_base = worker_base + c * CHUNK_ROWS
        pltpu.sync_copy(idx_hbm.at[pl.ds(row_base, CHUNK_ROWS)],
                        idx_v.at[pl.ds(p * CHUNK_ROWS, CHUNK_ROWS)])
        for j in range(GATHER_SPLIT):
            off = p * CHUNK_ROWS + j * GATHER_ROWS
            pltpu.async_copy(
                table_hbm.at[idx_v.at[pl.ds(off, GATHER_ROWS)]],
                g_v.at[pl.ds(off, GATHER_ROWS)],
                gsem,
            )

    fire_chunk(0, 0)

    def chunk_body(c, carry):
        p = lax.rem(c, 2)
        pn = 1 - p
        gbase = p * CHUNK_ROWS

        @pl.when(c > 0)
        def _():
            pltpu.make_async_copy(
                g_v.at[pl.ds(pn * CHUNK_ROWS, CHUNK_ROWS)],
                out_hbm.at[pl.ds(0, CHUNK_ROWS)],
                osem).wait()

        @pl.when(c < NUM_CHUNKS - 1)
        def _():
            fire_chunk(c + 1, pn)

        pltpu.make_async_copy(
            out_hbm.at[pl.ds(0, CHUNK_ROWS)],
            g_v.at[pl.ds(gbase, CHUNK_ROWS)],
            gsem).wait()

        def add_body(l, carry2):
            for r in range(D // 16):
                pv = pos_v[pl.ds(l * D + r * 16, 16)]
                for s in range(CHUNK_SEQ):
                    row = gbase + s * SEQ_LEN + l
                    plsc.addupdate(g_v.at[row, pl.ds(r * 16, 16)], pv)
            return carry2

        lax.fori_loop(0, SEQ_LEN, add_body, 0)

        row_base = worker_base + c * CHUNK_ROWS
        pltpu.async_copy(
            g_v.at[pl.ds(gbase, CHUNK_ROWS)],
            out_hbm.at[pl.ds(row_base, CHUNK_ROWS)],
            osem)
        return carry

    lax.fori_loop(0, NUM_CHUNKS, chunk_body, 0)

    pltpu.make_async_copy(
        g_v.at[pl.ds(CHUNK_ROWS, CHUNK_ROWS)],
        out_hbm.at[pl.ds(0, CHUNK_ROWS)],
        osem).wait()


@jax.jit
def _pos_embed(flat_idx, word_table, pos_flat):
    mesh = plsc.VectorSubcoreMesh(core_axis_name="c", subcore_axis_name="s")
    return pl.kernel(
        _body,
        mesh=mesh,
        compiler_params=pltpu.CompilerParams(use_tc_tiling_on_sc=False),
        out_type=jax.ShapeDtypeStruct((BATCH * SEQ_LEN, D), jnp.float32),
        scratch_types=[
            pltpu.VMEM((2 * CHUNK_ROWS,), jnp.int32),
            pltpu.VMEM((2 * CHUNK_ROWS, D), jnp.float32),
            pltpu.VMEM((SEQ_LEN * D,), jnp.float32),
            pltpu.SemaphoreType.DMA,
            pltpu.SemaphoreType.DMA,
        ],
    )(flat_idx, word_table, pos_flat)


def kernel(inputs, word_table, pos_table):
    flat_idx = inputs.reshape(-1)
    pos_flat = pos_table.reshape(-1)
    out = _pos_embed(flat_idx, word_table, pos_flat)
    return out.reshape(BATCH, SEQ_LEN, D)
